# baseline (speedup 1.0000x reference)
import functools

import numpy as np

import jax
import jax.numpy as jnp
from jax.experimental import pallas as pl
from jax.experimental.pallas import tpu as pltpu
from jax.sharding import Mesh, PartitionSpec as P

shard_map = jax.shard_map


def _cdiv(a: int, b: int) -> int:
    return (a + b - 1) // b


def _mlp_kernel(x_ref, w1_ref, b1_ref, w2_ref, b2_ref, o_ref):
    xb = x_ref[...].astype(jnp.bfloat16)
    w1b = w1_ref[...].astype(jnp.bfloat16)
    w2b = w2_ref[...].astype(jnp.bfloat16)
    h = jnp.dot(xb, w1b, preferred_element_type=jnp.float32)
    h = jnp.maximum(h + b1_ref[...], 0.0).astype(jnp.bfloat16)
    y = jnp.dot(h, w2b, preferred_element_type=jnp.float32)
    o_ref[...] = (y + b2_ref[...]).astype(o_ref.dtype)


def _forward_shard(x, w1_t, b1_f, w2_t, b2_f):
    b, n_feature = x.shape
    n_hidden, n_output = w2_t.shape

    tb = min(1024, max(8, _cdiv(b, 8) * 8))
    nb = _cdiv(b, tb)
    b_pad = nb * tb
    if b_pad != b:
        x = jnp.pad(x, ((0, b_pad - b), (0, 0)))

    out = pl.pallas_call(
        _mlp_kernel,
        out_shape=jax.ShapeDtypeStruct((b_pad, n_output), x.dtype),
        grid=(nb,),
        in_specs=[
            pl.BlockSpec((tb, n_feature), lambda i: (i, 0)),
            pl.BlockSpec((n_feature, n_hidden), lambda i: (0, 0)),
            pl.BlockSpec((1, n_hidden), lambda i: (0, 0)),
            pl.BlockSpec((n_hidden, n_output), lambda i: (0, 0)),
            pl.BlockSpec((1, n_output), lambda i: (0, 0)),
        ],
        out_specs=pl.BlockSpec((tb, n_output), lambda i: (i, 0)),
        compiler_params=pltpu.CompilerParams(
            dimension_semantics=("arbitrary",),
            vmem_limit_bytes=int(64 * 1024 * 1024 * 0.92)),
    )(x, w1_t, b1_f, w2_t, b2_f)

    if b_pad != b:
        out = out[:b]
    return out


@jax.jit
def kernel(x, w1_t, b1_r, w2_t, b2_r):
    b = x.shape[0]
    b1_f = b1_r.astype(jnp.float32)
    b2_f = b2_r.astype(jnp.float32)

    devs = jax.devices()
    ndev = 2 if (len(devs) >= 2 and b % 2 == 0) else 1
    if ndev == 1:
        return _forward_shard(x, w1_t, b1_f, w2_t, b2_f)

    mesh = Mesh(np.asarray(devs[:ndev]), ("b",))
    fwd = shard_map(
        _forward_shard,
        mesh=mesh,
        in_specs=(P("b", None), P(None, None), P(None, None),
                  P(None, None), P(None, None)),
        out_specs=P("b", None),
        check_vma=False,
    )
    return fwd(x, w1_t, b1_f, w2_t, b2_f)


# speedup vs baseline: 5.7485x; 5.7485x over previous
import functools

import jax
import jax.numpy as jnp
from jax.experimental import pallas as pl
from jax.experimental.pallas import tpu as pltpu


def _cdiv(a: int, b: int) -> int:
    return (a + b - 1) // b


def _mlp_kernel(x_ref, w1_ref, b1_ref, w2_ref, b2_ref, o_ref,
                w1b_ref, w2b_ref):
    i = pl.program_id(0)

    @pl.when(i == 0)
    def _():
        w1b_ref[...] = w1_ref[...].astype(jnp.bfloat16)
        w2b_ref[...] = w2_ref[...].astype(jnp.bfloat16)

    xb = x_ref[...].astype(jnp.bfloat16)
    h = jnp.dot(xb, w1b_ref[...], preferred_element_type=jnp.float32)
    h = jnp.maximum(h + b1_ref[...], 0.0).astype(jnp.bfloat16)
    y = jnp.dot(h, w2b_ref[...], preferred_element_type=jnp.float32)
    o_ref[...] = (y + b2_ref[...]).astype(o_ref.dtype)


@jax.jit
def kernel(x, w1_t, b1_r, w2_t, b2_r):
    b, n_feature = x.shape
    n_hidden, n_output = w2_t.shape
    b1_f = b1_r.astype(jnp.float32)
    b2_f = b2_r.astype(jnp.float32)

    tb = min(1024, max(8, _cdiv(b, 8) * 8))
    nb = _cdiv(b, tb)
    b_pad = nb * tb
    if b_pad != b:
        x = jnp.pad(x, ((0, b_pad - b), (0, 0)))

    out = pl.pallas_call(
        _mlp_kernel,
        out_shape=jax.ShapeDtypeStruct((b_pad, n_output), x.dtype),
        grid_spec=pltpu.PrefetchScalarGridSpec(
            num_scalar_prefetch=0,
            grid=(nb,),
            in_specs=[
                pl.BlockSpec((tb, n_feature), lambda i: (i, 0)),
                pl.BlockSpec((n_feature, n_hidden), lambda i: (0, 0)),
                pl.BlockSpec((1, n_hidden), lambda i: (0, 0)),
                pl.BlockSpec((n_hidden, n_output), lambda i: (0, 0)),
                pl.BlockSpec((1, n_output), lambda i: (0, 0)),
            ],
            out_specs=pl.BlockSpec((tb, n_output), lambda i: (i, 0)),
            scratch_shapes=[
                pltpu.VMEM((n_feature, n_hidden), jnp.bfloat16),
                pltpu.VMEM((n_hidden, n_output), jnp.bfloat16),
            ],
        ),
        compiler_params=pltpu.CompilerParams(
            dimension_semantics=("arbitrary",),
            vmem_limit_bytes=int(64 * 1024 * 1024 * 0.92)),
    )(x, w1_t, b1_f, w2_t, b2_f)

    if b_pad != b:
        out = out[:b]
    return out


# speedup vs baseline: 5.7577x; 1.0016x over previous
import functools

import jax
import jax.numpy as jnp
from jax.experimental import pallas as pl
from jax.experimental.pallas import tpu as pltpu

_TB = 512


def _cdiv(a: int, b: int) -> int:
    return (a + b - 1) // b


def _mlp_kernel(x_ref, w1_ref, b1_ref, w2_ref, b2_ref, o_ref):
    xb = x_ref[...].astype(jnp.bfloat16)
    w1b = w1_ref[...].astype(jnp.bfloat16)
    w2b = w2_ref[...].astype(jnp.bfloat16)
    h = jnp.dot(xb, w1b, preferred_element_type=jnp.float32)
    h = jnp.maximum(h + b1_ref[...], 0.0).astype(jnp.bfloat16)
    y = jnp.dot(h, w2b, preferred_element_type=jnp.float32)
    o_ref[...] = (y + b2_ref[...]).astype(o_ref.dtype)


@jax.jit
def kernel(x, w1_t, b1_r, w2_t, b2_r):
    b, n_feature = x.shape
    n_hidden, n_output = w2_t.shape
    b1_f = b1_r.astype(jnp.float32)
    b2_f = b2_r.astype(jnp.float32)

    tb = min(_TB, max(8, _cdiv(b, 8) * 8))
    nb = _cdiv(b, tb)
    b_pad = nb * tb
    if b_pad != b:
        x = jnp.pad(x, ((0, b_pad - b), (0, 0)))

    out = pl.pallas_call(
        _mlp_kernel,
        out_shape=jax.ShapeDtypeStruct((b_pad, n_output), x.dtype),
        grid=(nb,),
        in_specs=[
            pl.BlockSpec((tb, n_feature), lambda i: (i, 0)),
            pl.BlockSpec((n_feature, n_hidden), lambda i: (0, 0)),
            pl.BlockSpec((1, n_hidden), lambda i: (0, 0)),
            pl.BlockSpec((n_hidden, n_output), lambda i: (0, 0)),
            pl.BlockSpec((1, n_output), lambda i: (0, 0)),
        ],
        out_specs=pl.BlockSpec((tb, n_output), lambda i: (i, 0)),
        compiler_params=pltpu.CompilerParams(
            dimension_semantics=("arbitrary",),
            vmem_limit_bytes=int(64 * 1024 * 1024 * 0.92)),
    )(x, w1_t, b1_f, w2_t, b2_f)

    if b_pad != b:
        out = out[:b]
    return out


# speedup vs baseline: 5.8021x; 1.0077x over previous
import functools

import jax
import jax.numpy as jnp
from jax.experimental import pallas as pl
from jax.experimental.pallas import tpu as pltpu

_TB = 1024
_C1 = 4
_C2 = 4


def _cdiv(a: int, b: int) -> int:
    return (a + b - 1) // b


def _make_kernel(n_feature, n_hidden, n_output, c1, c2):
    f_ch = n_feature // c1
    h_ch = n_hidden // c2

    def _mlp_kernel(x_ref, w1_ref, b1_ref, w2_ref, b2_ref, o_ref,
                    w1b_ref, w2b_ref, stg1_ref, stg2_ref, sem1, sem2):
        i = pl.program_id(0)

        def w1_copy(c):
            return pltpu.make_async_copy(
                w1_ref.at[pl.ds(c * f_ch, f_ch), :], stg1_ref.at[c],
                sem1.at[c])

        def w2_copy(c):
            return pltpu.make_async_copy(
                w2_ref.at[pl.ds(c * h_ch, h_ch), :], stg2_ref.at[c],
                sem2.at[c])

        @pl.when(i == 0)
        def _():
            for c in range(c1):
                w1_copy(c).start()
            for c in range(c2):
                w2_copy(c).start()
            xb = x_ref[...].astype(jnp.bfloat16)
            h = None
            for c in range(c1):
                w1_copy(c).wait()
                wc = stg1_ref[c].astype(jnp.bfloat16)
                w1b_ref[pl.ds(c * f_ch, f_ch), :] = wc
                part = jnp.dot(xb[:, c * f_ch:(c + 1) * f_ch], wc,
                               preferred_element_type=jnp.float32)
                h = part if h is None else h + part
            hb = jnp.maximum(h + b1_ref[...], 0.0).astype(jnp.bfloat16)
            y = None
            for c in range(c2):
                w2_copy(c).wait()
                wc = stg2_ref[c].astype(jnp.bfloat16)
                w2b_ref[pl.ds(c * h_ch, h_ch), :] = wc
                part = jnp.dot(hb[:, c * h_ch:(c + 1) * h_ch], wc,
                               preferred_element_type=jnp.float32)
                y = part if y is None else y + part
            o_ref[...] = (y + b2_ref[...]).astype(o_ref.dtype)

        @pl.when(i > 0)
        def _():
            xb = x_ref[...].astype(jnp.bfloat16)
            h = jnp.dot(xb, w1b_ref[...], preferred_element_type=jnp.float32)
            hb = jnp.maximum(h + b1_ref[...], 0.0).astype(jnp.bfloat16)
            y = jnp.dot(hb, w2b_ref[...], preferred_element_type=jnp.float32)
            o_ref[...] = (y + b2_ref[...]).astype(o_ref.dtype)

    return _mlp_kernel


@functools.partial(jax.jit, static_argnames=("interpret",))
def kernel(x, w1_t, b1_r, w2_t, b2_r, interpret=False):
    b, n_feature = x.shape
    n_hidden, n_output = w2_t.shape
    b1_f = b1_r.astype(jnp.float32)
    b2_f = b2_r.astype(jnp.float32)

    c1 = _C1 if n_feature % _C1 == 0 else 1
    c2 = _C2 if n_hidden % _C2 == 0 else 1

    tb = min(_TB, max(8, _cdiv(b, 8) * 8))
    nb = _cdiv(b, tb)
    b_pad = nb * tb
    if b_pad != b:
        x = jnp.pad(x, ((0, b_pad - b), (0, 0)))

    out = pl.pallas_call(
        _make_kernel(n_feature, n_hidden, n_output, c1, c2),
        out_shape=jax.ShapeDtypeStruct((b_pad, n_output), x.dtype),
        grid_spec=pltpu.PrefetchScalarGridSpec(
            num_scalar_prefetch=0,
            grid=(nb,),
            in_specs=[
                pl.BlockSpec((tb, n_feature), lambda i: (i, 0)),
                pl.BlockSpec(memory_space=pl.ANY),
                pl.BlockSpec((1, n_hidden), lambda i: (0, 0)),
                pl.BlockSpec(memory_space=pl.ANY),
                pl.BlockSpec((1, n_output), lambda i: (0, 0)),
            ],
            out_specs=pl.BlockSpec((tb, n_output), lambda i: (i, 0)),
            scratch_shapes=[
                pltpu.VMEM((n_feature, n_hidden), jnp.bfloat16),
                pltpu.VMEM((n_hidden, n_output), jnp.bfloat16),
                pltpu.VMEM((c1, n_feature // c1, n_hidden), jnp.float32),
                pltpu.VMEM((c2, n_hidden // c2, n_output), jnp.float32),
                pltpu.SemaphoreType.DMA((c1,)),
                pltpu.SemaphoreType.DMA((c2,)),
            ],
        ),
        compiler_params=pltpu.CompilerParams(
            dimension_semantics=("arbitrary",),
            vmem_limit_bytes=int(64 * 1024 * 1024 * 0.92)),
        interpret=interpret,
    )(x, w1_t, b1_f, w2_t, b2_f)

    if b_pad != b:
        out = out[:b]
    return out


# speedup vs baseline: 5.8151x; 1.0022x over previous
import functools

import jax
import jax.numpy as jnp
from jax.experimental import pallas as pl
from jax.experimental.pallas import tpu as pltpu

_TB = 1024


def _cdiv(a: int, b: int) -> int:
    return (a + b - 1) // b


def _mlp_kernel(x_ref, w1_ref, b1_ref, w2_ref, b2_ref, o_ref):
    xb = x_ref[...].astype(jnp.bfloat16)
    w1b = w1_ref[...].astype(jnp.bfloat16)
    w2b = w2_ref[...].astype(jnp.bfloat16)
    h = jnp.dot(xb, w1b, preferred_element_type=jnp.float32)
    hb = jnp.maximum(h + b1_ref[...], 0.0).astype(jnp.bfloat16)
    y = jnp.dot(hb, w2b, preferred_element_type=jnp.float32)
    o_ref[...] = (y + b2_ref[...]).astype(o_ref.dtype)


@functools.partial(jax.jit, static_argnames=("interpret",))
def kernel(x, w1_t, b1_r, w2_t, b2_r, interpret=False):
    b, n_feature = x.shape
    n_hidden, n_output = w2_t.shape
    b1_f = b1_r.astype(jnp.float32)
    b2_f = b2_r.astype(jnp.float32)

    tb = min(_TB, max(8, _cdiv(b, 8) * 8))
    nb = _cdiv(b, tb)
    b_pad = nb * tb
    if b_pad != b:
        x = jnp.pad(x, ((0, b_pad - b), (0, 0)))

    out = pl.pallas_call(
        _mlp_kernel,
        out_shape=jax.ShapeDtypeStruct((b_pad, n_output), x.dtype),
        grid=(nb,),
        in_specs=[
            pl.BlockSpec((tb, n_feature), lambda i: (i, 0)),
            pl.BlockSpec((n_feature, n_hidden), lambda i: (0, 0)),
            pl.BlockSpec((1, n_hidden), lambda i: (0, 0)),
            pl.BlockSpec((n_hidden, n_output), lambda i: (0, 0)),
            pl.BlockSpec((1, n_output), lambda i: (0, 0)),
        ],
        out_specs=pl.BlockSpec((tb, n_output), lambda i: (i, 0)),
        compiler_params=pltpu.CompilerParams(
            dimension_semantics=("arbitrary",),
            vmem_limit_bytes=int(64 * 1024 * 1024 * 0.92)),
        interpret=interpret,
    )(x, w1_t, b1_f, w2_t, b2_f)

    if b_pad != b:
        out = out[:b]
    return out


# speedup vs baseline: 5.8273x; 1.0021x over previous
import functools

import jax
import jax.numpy as jnp
from jax.experimental import pallas as pl
from jax.experimental.pallas import tpu as pltpu

_TB = 1024


def _cdiv(a: int, b: int) -> int:
    return (a + b - 1) // b


def _mlp_kernel(x_ref, w1_ref, b1_ref, w2_ref, b2_ref, o_ref):
    xb = x_ref[...].astype(jnp.bfloat16)
    w1b = w1_ref[...].astype(jnp.bfloat16)
    w2b = w2_ref[...].astype(jnp.bfloat16)
    h = jnp.dot(xb, w1b, preferred_element_type=jnp.float32)
    hb = jnp.maximum(h + b1_ref[...], 0.0).astype(jnp.bfloat16)
    y = jnp.dot(hb, w2b, preferred_element_type=jnp.float32)
    o_ref[...] = (y + b2_ref[...]).astype(o_ref.dtype)


@jax.jit
def kernel(x, w1_t, b1_r, w2_t, b2_r):
    b, n_feature = x.shape
    n_hidden, n_output = w2_t.shape
    b1_f = b1_r.astype(jnp.float32)
    b2_f = b2_r.astype(jnp.float32)

    tb = min(_TB, max(8, _cdiv(b, 8) * 8))
    nb = _cdiv(b, tb)
    b_pad = nb * tb
    if b_pad != b:
        x = jnp.pad(x, ((0, b_pad - b), (0, 0)))

    out = pl.pallas_call(
        _mlp_kernel,
        out_shape=jax.ShapeDtypeStruct((b_pad, n_output), x.dtype),
        grid=(nb,),
        in_specs=[
            pl.BlockSpec((tb, n_feature), lambda i: (i, 0)),
            pl.BlockSpec((n_feature, n_hidden), lambda i: (0, 0)),
            pl.BlockSpec((1, n_hidden), lambda i: (0, 0)),
            pl.BlockSpec((n_hidden, n_output), lambda i: (0, 0)),
            pl.BlockSpec((1, n_output), lambda i: (0, 0)),
        ],
        out_specs=pl.BlockSpec((tb, n_output), lambda i: (i, 0)),
        compiler_params=pltpu.CompilerParams(
            dimension_semantics=("arbitrary",),
            vmem_limit_bytes=int(64 * 1024 * 1024 * 0.92)),
    )(x, w1_t, b1_f, w2_t, b2_f)

    if b_pad != b:
        out = out[:b]
    return out
